# baseline (speedup 1.0000x reference)
import functools

import jax
import jax.numpy as jnp
from jax import lax
from jax.experimental import pallas as pl
from jax.experimental.pallas import tpu as pltpu
from jax.experimental.pallas import tpu_sc as plsc

N = 50000
D = 128
S = 6
EL = 10000
L = 4
NE = 14

NP = 50688
BN = 512
GRID = NP // BN

NCHUNK = 6
CH = NP // NCHUNK
NT = 16
CHT = CH // NT
T_TILE = 7168
NB = T_TILE // 128
E_TOT = 2 * S * N + 2 * EL


def _gemm_body(feat_ref, wctr_ref, wedge_ref, yctr_ref, yedge_ref):
    f = feat_ref[...]
    yctr_ref[...] = lax.dot_general(f, wctr_ref[...], (((1,), (1,)), ((), ())),
                                    preferred_element_type=jnp.float32)
    yedge_ref[...] = jnp.dot(f, wedge_ref[...], preferred_element_type=jnp.float32)


_gemm = pl.pallas_call(
    _gemm_body,
    grid=(GRID,),
    in_specs=[
        pl.BlockSpec((BN, D), lambda i: (i, 0)),
        pl.BlockSpec((D, D), lambda i: (0, 0)),
        pl.BlockSpec((D, NE * D), lambda i: (0, 0)),
    ],
    out_specs=[
        pl.BlockSpec((BN, D), lambda i: (i, 0)),
        pl.BlockSpec((BN, NE * D), lambda i: (i, 0)),
    ],
    out_shape=[
        jax.ShapeDtypeStruct((NP, D), jnp.float32),
        jax.ShapeDtypeStruct((NP, NE * D), jnp.float32),
    ],
)


def _sc_body(yctr, yflat, uloc, gvl, out, uidx, gvidx, rows0, rows1, sem0, sem1, acc):
    cid = lax.axis_index("c")
    sid = lax.axis_index("s")
    for cc in range(NCHUNK // 2):
        c = cid * (NCHUNK // 2) + cc
        r0 = c * CH + sid * CHT
        pltpu.sync_copy(yctr.at[pl.ds(r0, CHT)], acc.at[pl.ds(sid * CHT, CHT)])
        pltpu.sync_copy(uloc.at[c * NT + sid], uidx)
        pltpu.sync_copy(gvl.at[c * NT + sid], gvidx)
        plsc.subcore_barrier()
        pltpu.async_copy(yflat.at[gvidx.at[0]], rows0, sem0)
        pltpu.async_copy(yflat.at[gvidx.at[1]], rows1, sem1)

        def body(m, carry):
            j = 2 * m
            pltpu.make_async_copy(yflat.at[gvidx.at[j]], rows0, sem0).wait()
            pltpu.sync_copy(rows0, acc.at[uidx.at[j]], add=True)
            pltpu.async_copy(yflat.at[gvidx.at[j + 2]], rows0, sem0)
            pltpu.make_async_copy(yflat.at[gvidx.at[j + 1]], rows1, sem1).wait()
            pltpu.sync_copy(rows1, acc.at[uidx.at[j + 1]], add=True)
            pltpu.async_copy(yflat.at[gvidx.at[j + 3]], rows1, sem1)
            return carry

        lax.fori_loop(0, NB // 2 - 1, body, 0)
        pltpu.make_async_copy(yflat.at[gvidx.at[NB - 2]], rows0, sem0).wait()
        pltpu.sync_copy(rows0, acc.at[uidx.at[NB - 2]], add=True)
        pltpu.make_async_copy(yflat.at[gvidx.at[NB - 1]], rows1, sem1).wait()
        pltpu.sync_copy(rows1, acc.at[uidx.at[NB - 1]], add=True)
        plsc.subcore_barrier()
        pltpu.sync_copy(acc.at[pl.ds(sid * CHT, CHT)], out.at[pl.ds(r0, CHT)])
        plsc.subcore_barrier()


_sc_scatter = pl.kernel(
    _sc_body,
    out_type=jax.ShapeDtypeStruct((NP, D), jnp.float32),
    mesh=plsc.VectorSubcoreMesh(core_axis_name="c", subcore_axis_name="s",
                                num_cores=2, num_subcores=NT),
    scratch_types=[
        pltpu.VMEM((NB, 128), jnp.int32),
        pltpu.VMEM((NB, 128), jnp.int32),
        pltpu.VMEM((128, D), jnp.float32),
        pltpu.VMEM((128, D), jnp.float32),
        pltpu.SemaphoreType.DMA,
        pltpu.SemaphoreType.DMA,
        pltpu.VMEM_SHARED((CH + 16, D), jnp.float32),
    ],
)


def _epi_body(scat_ref, res_ref, w2_ref, g1w_ref, g1b_ref, g2w_ref, g2b_ref, out_ref):
    x = scat_ref[...]
    mu = jnp.mean(x, axis=1, keepdims=True)
    var = jnp.mean((x - mu) * (x - mu), axis=1, keepdims=True)
    a = (x - mu) * lax.rsqrt(var + 1e-5) * g1w_ref[...] + g1b_ref[...]
    a = jnp.maximum(a, 0.0)
    y = lax.dot_general(a, w2_ref[...], (((1,), (1,)), ((), ())),
                        preferred_element_type=jnp.float32)
    mu2 = jnp.mean(y, axis=1, keepdims=True)
    var2 = jnp.mean((y - mu2) * (y - mu2), axis=1, keepdims=True)
    z = (y - mu2) * lax.rsqrt(var2 + 1e-5) * g2w_ref[...] + g2b_ref[...]
    out_ref[...] = jnp.maximum(z + res_ref[...], 0.0)


def _make_epilogue(rows, bn):
    return pl.pallas_call(
        _epi_body,
        grid=(rows // bn,),
        in_specs=[
            pl.BlockSpec((bn, D), lambda i: (i, 0)),
            pl.BlockSpec((bn, D), lambda i: (i, 0)),
            pl.BlockSpec((D, D), lambda i: (0, 0)),
            pl.BlockSpec((1, D), lambda i: (0, 0)),
            pl.BlockSpec((1, D), lambda i: (0, 0)),
            pl.BlockSpec((1, D), lambda i: (0, 0)),
            pl.BlockSpec((1, D), lambda i: (0, 0)),
        ],
        out_specs=pl.BlockSpec((bn, D), lambda i: (i, 0)),
        out_shape=jax.ShapeDtypeStruct((rows, D), jnp.float32),
    )


_epi_mid = _make_epilogue(NP, BN)
_epi_last = _make_epilogue(N, 400)


def _build_edge_schedule(pre_u, pre_v, suc_u, suc_v, left_u, left_v, right_u, right_v):
    u_all = jnp.concatenate([pre_u.reshape(-1), suc_u.reshape(-1), left_u, right_u])
    v_all = jnp.concatenate([pre_v.reshape(-1), suc_v.reshape(-1), left_v, right_v])
    t_all = jnp.concatenate([
        jnp.repeat(jnp.arange(S, dtype=jnp.int32), N),
        jnp.repeat(jnp.arange(S, 2 * S, dtype=jnp.int32), N),
        jnp.full((EL,), 12, jnp.int32),
        jnp.full((EL,), 13, jnp.int32),
    ])
    gv_all = v_all * NE + t_all
    chunk = u_all // CH
    chunk_s, u_s, gv_s = lax.sort((chunk, u_all, gv_all), num_keys=1)
    off = jnp.searchsorted(chunk_s, jnp.arange(NCHUNK, dtype=chunk_s.dtype))
    cnt = jnp.concatenate([off[1:], jnp.array([E_TOT], off.dtype)]) - off
    ct = (cnt + NT - 1) // NT
    c_ix = jnp.arange(NCHUNK, dtype=jnp.int32)[:, None, None]
    j_ix = jnp.arange(NT, dtype=jnp.int32)[None, :, None]
    k_ix = jnp.arange(T_TILE, dtype=jnp.int32)[None, None, :]
    local = j_ix * ct[:, None, None].astype(jnp.int32) + k_ix
    valid = (k_ix < ct[:, None, None]) & (local < cnt[:, None, None])
    g = jnp.clip(off[:, None, None].astype(jnp.int32) + local, 0, E_TOT - 1)
    u_loc = jnp.where(valid, u_s[g] - c_ix * CH, CH).astype(jnp.int32)
    gv_pad = jnp.where(valid, gv_s[g], 0).astype(jnp.int32)
    return (u_loc.reshape(NCHUNK * NT, NB, 128), gv_pad.reshape(NCHUNK * NT, NB, 128))


def kernel(feat, pre_u, pre_v, suc_u, suc_v, left_u, left_v, right_u, right_v,
           W_ctr, W_pre, W_suc, W_left, W_right, W_ctr2,
           gn1_w, gn1_b, gn2_w, gn2_b):
    u_loc, gv_pad = _build_edge_schedule(pre_u, pre_v, suc_u, suc_v,
                                         left_u, left_v, right_u, right_v)
    ws = jnp.concatenate([W_pre, W_suc, W_left[:, None], W_right[:, None]], axis=1)
    wedge = ws.transpose(0, 3, 1, 2).reshape(L, D, NE * D)

    featp = jnp.pad(feat, ((0, NP - N), (0, 0)))
    res = featp
    for i in range(L):
        yctr, yedge = _gemm(featp, W_ctr[i], wedge[i])
        scat = _sc_scatter(yctr, yedge.reshape(NP * NE, D), u_loc, gv_pad)
        epi = _epi_mid if i < L - 1 else _epi_last
        featp = epi(scat, res, W_ctr2[i],
                    gn1_w[i][None], gn1_b[i][None], gn2_w[i][None], gn2_b[i][None])
        res = featp
    return featp


# speedup vs baseline: 1.1037x; 1.1037x over previous
import functools

import jax
import jax.numpy as jnp
from jax import lax
from jax.experimental import pallas as pl
from jax.experimental.pallas import tpu as pltpu
from jax.experimental.pallas import tpu_sc as plsc

N = 50000
D = 128
S = 6
EL = 10000
L = 4
NE = 14

NP = 50688
BN = 512
GRID = NP // BN

NCHUNK = 6
CH = NP // NCHUNK
NT = 16
CHT = CH // NT
T_TILE = 7040
NB = T_TILE // 128
E_TOT = 2 * S * N + 2 * EL


def _gemm_body(feat_ref, wctr_ref, wedge_ref, yctr_ref, yedge_ref):
    f = feat_ref[...]
    yctr_ref[...] = lax.dot_general(f, wctr_ref[...], (((1,), (1,)), ((), ())),
                                    preferred_element_type=jnp.float32)
    yedge_ref[...] = jnp.dot(f, wedge_ref[...], preferred_element_type=jnp.float32)


_gemm = pl.pallas_call(
    _gemm_body,
    grid=(GRID,),
    in_specs=[
        pl.BlockSpec((BN, D), lambda i: (i, 0)),
        pl.BlockSpec((D, D), lambda i: (0, 0)),
        pl.BlockSpec((D, NE * D), lambda i: (0, 0)),
    ],
    out_specs=[
        pl.BlockSpec((BN, D), lambda i: (i, 0)),
        pl.BlockSpec((BN, NE * D), lambda i: (i, 0)),
    ],
    out_shape=[
        jax.ShapeDtypeStruct((NP, D), jnp.float32),
        jax.ShapeDtypeStruct((NP, NE * D), jnp.float32),
    ],
)


def _sc_body(yctr, yflat, uloc, gvl, out, uidx, gvidx, rows, sem, acc):
    cid = lax.axis_index("c")
    sid = lax.axis_index("s")
    for cc in range(NCHUNK // 2):
        c = cid * (NCHUNK // 2) + cc
        r0 = c * CH + sid * CHT
        pltpu.sync_copy(yctr.at[pl.ds(r0, CHT)], acc.at[pl.ds(sid * CHT, CHT)])
        pltpu.sync_copy(uloc.at[c * NT + sid], uidx)
        pltpu.sync_copy(gvl.at[c * NT + sid], gvidx)
        plsc.subcore_barrier()

        def body(j, carry):
            pltpu.async_copy(yflat.at[gvidx.at[j]], rows, sem).wait()
            pltpu.sync_copy(rows, acc.at[uidx.at[j]], add=True)
            return carry

        lax.fori_loop(0, NB, body, 0)
        plsc.subcore_barrier()
        pltpu.sync_copy(acc.at[pl.ds(sid * CHT, CHT)], out.at[pl.ds(r0, CHT)])
        plsc.subcore_barrier()


_sc_scatter = pl.kernel(
    _sc_body,
    out_type=jax.ShapeDtypeStruct((NP, D), jnp.float32),
    mesh=plsc.VectorSubcoreMesh(core_axis_name="c", subcore_axis_name="s",
                                num_cores=2, num_subcores=NT),
    scratch_types=[
        pltpu.VMEM((NB, 128), jnp.int32),
        pltpu.VMEM((NB, 128), jnp.int32),
        pltpu.VMEM((128, D), jnp.float32),
        pltpu.SemaphoreType.DMA,
        pltpu.VMEM_SHARED((CH + 16, D), jnp.float32),
    ],
)


def _epi_body(scat_ref, res_ref, w2_ref, g1w_ref, g1b_ref, g2w_ref, g2b_ref, out_ref):
    x = scat_ref[...]
    mu = jnp.mean(x, axis=1, keepdims=True)
    var = jnp.mean((x - mu) * (x - mu), axis=1, keepdims=True)
    a = (x - mu) * lax.rsqrt(var + 1e-5) * g1w_ref[...] + g1b_ref[...]
    a = jnp.maximum(a, 0.0)
    y = lax.dot_general(a, w2_ref[...], (((1,), (1,)), ((), ())),
                        preferred_element_type=jnp.float32)
    mu2 = jnp.mean(y, axis=1, keepdims=True)
    var2 = jnp.mean((y - mu2) * (y - mu2), axis=1, keepdims=True)
    z = (y - mu2) * lax.rsqrt(var2 + 1e-5) * g2w_ref[...] + g2b_ref[...]
    out_ref[...] = jnp.maximum(z + res_ref[...], 0.0)


def _make_epilogue(rows, bn):
    return pl.pallas_call(
        _epi_body,
        grid=(rows // bn,),
        in_specs=[
            pl.BlockSpec((bn, D), lambda i: (i, 0)),
            pl.BlockSpec((bn, D), lambda i: (i, 0)),
            pl.BlockSpec((D, D), lambda i: (0, 0)),
            pl.BlockSpec((1, D), lambda i: (0, 0)),
            pl.BlockSpec((1, D), lambda i: (0, 0)),
            pl.BlockSpec((1, D), lambda i: (0, 0)),
            pl.BlockSpec((1, D), lambda i: (0, 0)),
        ],
        out_specs=pl.BlockSpec((bn, D), lambda i: (i, 0)),
        out_shape=jax.ShapeDtypeStruct((rows, D), jnp.float32),
    )


_epi_mid = _make_epilogue(NP, BN)
_epi_last = _make_epilogue(N, 400)


def _build_edge_schedule(pre_u, pre_v, suc_u, suc_v, left_u, left_v, right_u, right_v):
    u_all = jnp.concatenate([pre_u.reshape(-1), suc_u.reshape(-1), left_u, right_u])
    v_all = jnp.concatenate([pre_v.reshape(-1), suc_v.reshape(-1), left_v, right_v])
    t_all = jnp.concatenate([
        jnp.repeat(jnp.arange(S, dtype=jnp.int32), N),
        jnp.repeat(jnp.arange(S, 2 * S, dtype=jnp.int32), N),
        jnp.full((EL,), 12, jnp.int32),
        jnp.full((EL,), 13, jnp.int32),
    ])
    gv_all = v_all * NE + t_all
    chunk = u_all // CH
    chunk_s, u_s, gv_s = lax.sort((chunk, u_all, gv_all), num_keys=1)
    off = jnp.searchsorted(chunk_s, jnp.arange(NCHUNK, dtype=chunk_s.dtype))
    cnt = jnp.concatenate([off[1:], jnp.array([E_TOT], off.dtype)]) - off
    ct = (cnt + NT - 1) // NT
    c_ix = jnp.arange(NCHUNK, dtype=jnp.int32)[:, None, None]
    j_ix = jnp.arange(NT, dtype=jnp.int32)[None, :, None]
    k_ix = jnp.arange(T_TILE, dtype=jnp.int32)[None, None, :]
    local = j_ix * ct[:, None, None].astype(jnp.int32) + k_ix
    valid = (k_ix < ct[:, None, None]) & (local < cnt[:, None, None])
    g = jnp.clip(off[:, None, None].astype(jnp.int32) + local, 0, E_TOT - 1)
    u_loc = jnp.where(valid, u_s[g] - c_ix * CH, CH).astype(jnp.int32)
    gv_pad = jnp.where(valid, gv_s[g], 0).astype(jnp.int32)
    return (u_loc.reshape(NCHUNK * NT, NB, 128), gv_pad.reshape(NCHUNK * NT, NB, 128))


def kernel(feat, pre_u, pre_v, suc_u, suc_v, left_u, left_v, right_u, right_v,
           W_ctr, W_pre, W_suc, W_left, W_right, W_ctr2,
           gn1_w, gn1_b, gn2_w, gn2_b):
    u_loc, gv_pad = _build_edge_schedule(pre_u, pre_v, suc_u, suc_v,
                                         left_u, left_v, right_u, right_v)
    ws = jnp.concatenate([W_pre, W_suc, W_left[:, None], W_right[:, None]], axis=1)
    wedge = ws.transpose(0, 3, 1, 2).reshape(L, D, NE * D)

    featp = jnp.pad(feat, ((0, NP - N), (0, 0)))
    res = featp
    for i in range(L):
        yctr, yedge = _gemm(featp, W_ctr[i], wedge[i])
        scat = _sc_scatter(yctr, yedge.reshape(NP * NE, D), u_loc, gv_pad)
        epi = _epi_mid if i < L - 1 else _epi_last
        featp = epi(scat, res, W_ctr2[i],
                    gn1_w[i][None], gn1_b[i][None], gn2_w[i][None], gn2_b[i][None])
        res = featp
    return featp


# speedup vs baseline: 1.2825x; 1.1620x over previous
import functools

import jax
import jax.numpy as jnp
from jax import lax
from jax.experimental import pallas as pl
from jax.experimental.pallas import tpu as pltpu
from jax.experimental.pallas import tpu_sc as plsc

N = 50000
D = 128
S = 6
EL = 10000
L = 4
NE = 14

NP = 50688
BN = 512
GRID = NP // BN

NCHUNK = 6
CH = NP // NCHUNK
NT = 16
CHT = CH // NT
T_TILE = 6912
NB = T_TILE // 128
E_TOT = 2 * S * N + 2 * EL


def _gemm_body(feat_ref, wctr_ref, wedge_ref, yctr_ref, yedge_ref):
    f = feat_ref[...]
    yctr_ref[...] = lax.dot_general(f, wctr_ref[...], (((1,), (1,)), ((), ())),
                                    preferred_element_type=jnp.float32)
    yedge_ref[...] = jnp.dot(f, wedge_ref[...], preferred_element_type=jnp.float32)


_gemm = pl.pallas_call(
    _gemm_body,
    grid=(GRID,),
    in_specs=[
        pl.BlockSpec((BN, D), lambda i: (i, 0)),
        pl.BlockSpec((D, D), lambda i: (0, 0)),
        pl.BlockSpec((D, NE * D), lambda i: (0, 0)),
    ],
    out_specs=[
        pl.BlockSpec((BN, D), lambda i: (i, 0)),
        pl.BlockSpec((BN, NE * D), lambda i: (i, 0)),
    ],
    out_shape=[
        jax.ShapeDtypeStruct((NP, D), jnp.float32),
        jax.ShapeDtypeStruct((NP, NE * D), jnp.float32),
    ],
)


def _sc_body(yctr, yflat, uloc, gvl, out, uidx, gvidx, rows, sem, acc):
    cid = lax.axis_index("c")
    sid = lax.axis_index("s")
    for cc in range(NCHUNK // 2):
        c = cid * (NCHUNK // 2) + cc
        r0 = c * CH + sid * CHT
        pltpu.sync_copy(yctr.at[pl.ds(r0, CHT)], acc.at[pl.ds(sid * CHT, CHT)])
        pltpu.sync_copy(uloc.at[c * NT + sid], uidx)
        pltpu.sync_copy(gvl.at[c * NT + sid], gvidx)
        plsc.subcore_barrier()

        def body(j, carry):
            pltpu.async_copy(yflat.at[gvidx.at[j]], rows, sem).wait()
            pltpu.sync_copy(rows, acc.at[uidx.at[j]], add=True)
            return carry

        lax.fori_loop(0, NB, body, 0)
        plsc.subcore_barrier()
        pltpu.sync_copy(acc.at[pl.ds(sid * CHT, CHT)], out.at[pl.ds(r0, CHT)])
        plsc.subcore_barrier()


_sc_scatter = pl.kernel(
    _sc_body,
    out_type=jax.ShapeDtypeStruct((NP, D), jnp.float32),
    mesh=plsc.VectorSubcoreMesh(core_axis_name="c", subcore_axis_name="s",
                                num_cores=2, num_subcores=NT),
    scratch_types=[
        pltpu.VMEM((NB, 128), jnp.int32),
        pltpu.VMEM((NB, 128), jnp.int32),
        pltpu.VMEM((128, D), jnp.float32),
        pltpu.SemaphoreType.DMA,
        pltpu.VMEM_SHARED((CH + 16, D), jnp.float32),
    ],
)


def _epi_body(scat_ref, res_ref, w2_ref, g1w_ref, g1b_ref, g2w_ref, g2b_ref, out_ref):
    x = scat_ref[...]
    mu = jnp.mean(x, axis=1, keepdims=True)
    var = jnp.mean((x - mu) * (x - mu), axis=1, keepdims=True)
    a = (x - mu) * lax.rsqrt(var + 1e-5) * g1w_ref[...] + g1b_ref[...]
    a = jnp.maximum(a, 0.0)
    y = lax.dot_general(a, w2_ref[...], (((1,), (1,)), ((), ())),
                        preferred_element_type=jnp.float32)
    mu2 = jnp.mean(y, axis=1, keepdims=True)
    var2 = jnp.mean((y - mu2) * (y - mu2), axis=1, keepdims=True)
    z = (y - mu2) * lax.rsqrt(var2 + 1e-5) * g2w_ref[...] + g2b_ref[...]
    out_ref[...] = jnp.maximum(z + res_ref[...], 0.0)


def _make_epilogue(rows, bn):
    return pl.pallas_call(
        _epi_body,
        grid=(rows // bn,),
        in_specs=[
            pl.BlockSpec((bn, D), lambda i: (i, 0)),
            pl.BlockSpec((bn, D), lambda i: (i, 0)),
            pl.BlockSpec((D, D), lambda i: (0, 0)),
            pl.BlockSpec((1, D), lambda i: (0, 0)),
            pl.BlockSpec((1, D), lambda i: (0, 0)),
            pl.BlockSpec((1, D), lambda i: (0, 0)),
            pl.BlockSpec((1, D), lambda i: (0, 0)),
        ],
        out_specs=pl.BlockSpec((bn, D), lambda i: (i, 0)),
        out_shape=jax.ShapeDtypeStruct((rows, D), jnp.float32),
    )


_epi_mid = _make_epilogue(NP, BN)
_epi_last = _make_epilogue(N, 400)


def _build_edge_schedule(pre_u, pre_v, suc_u, suc_v, left_u, left_v, right_u, right_v):
    u_all = jnp.concatenate([pre_u.reshape(-1), suc_u.reshape(-1), left_u, right_u])
    v_all = jnp.concatenate([pre_v.reshape(-1), suc_v.reshape(-1), left_v, right_v])
    t_all = jnp.concatenate([
        jnp.repeat(jnp.arange(S, dtype=jnp.int32), N),
        jnp.repeat(jnp.arange(S, 2 * S, dtype=jnp.int32), N),
        jnp.full((EL,), 12, jnp.int32),
        jnp.full((EL,), 13, jnp.int32),
    ])
    gv_all = v_all * NE + t_all
    chunk = u_all // CH
    chunk_s, u_s, gv_s = lax.sort((chunk, u_all, gv_all), num_keys=1)
    off = jnp.searchsorted(chunk_s, jnp.arange(NCHUNK, dtype=chunk_s.dtype))
    cnt = jnp.concatenate([off[1:], jnp.array([E_TOT], off.dtype)]) - off
    ct = (cnt + NT - 1) // NT
    c_ix = jnp.arange(NCHUNK, dtype=jnp.int32)[:, None, None]
    j_ix = jnp.arange(NT, dtype=jnp.int32)[None, :, None]
    k_ix = jnp.arange(T_TILE, dtype=jnp.int32)[None, None, :]
    local = j_ix * ct[:, None, None].astype(jnp.int32) + k_ix
    valid = (k_ix < ct[:, None, None]) & (local < cnt[:, None, None])
    g = jnp.clip(off[:, None, None].astype(jnp.int32) + local, 0, E_TOT - 1)
    u_loc = jnp.where(valid, u_s[g] - c_ix * CH, CH).astype(jnp.int32)
    gv_pad = jnp.where(valid, gv_s[g], 0).astype(jnp.int32)
    return (u_loc.reshape(NCHUNK * NT, NB, 128), gv_pad.reshape(NCHUNK * NT, NB, 128))


def kernel(feat, pre_u, pre_v, suc_u, suc_v, left_u, left_v, right_u, right_v,
           W_ctr, W_pre, W_suc, W_left, W_right, W_ctr2,
           gn1_w, gn1_b, gn2_w, gn2_b):
    u_loc, gv_pad = _build_edge_schedule(pre_u, pre_v, suc_u, suc_v,
                                         left_u, left_v, right_u, right_v)
    ws = jnp.concatenate([W_pre, W_suc, W_left[:, None], W_right[:, None]], axis=1)
    wedge = ws.transpose(0, 3, 1, 2).reshape(L, D, NE * D)

    featp = jnp.pad(feat, ((0, NP - N), (0, 0)))
    res = featp
    for i in range(L):
        yctr, yedge = _gemm(featp, W_ctr[i], wedge[i])
        scat = _sc_scatter(yctr, yedge.reshape(NP * NE, D), u_loc, gv_pad)
        epi = _epi_mid if i < L - 1 else _epi_last
        featp = epi(scat, res, W_ctr2[i],
                    gn1_w[i][None], gn1_b[i][None], gn2_w[i][None], gn2_b[i][None])
        res = featp
    return featp


# speedup vs baseline: 1.5358x; 1.1976x over previous
import functools

import jax
import jax.numpy as jnp
from jax import lax
from jax.experimental import pallas as pl
from jax.experimental.pallas import tpu as pltpu
from jax.experimental.pallas import tpu_sc as plsc

N = 50000
D = 128
S = 6
EL = 10000
L = 4
NE = 14

NP = 50688
BN = 512
GRID = NP // BN

NCHUNK = 6
CH = NP // NCHUNK
NT = 16
CHT = CH // NT
T_TILE = 6784
NB = T_TILE // 128
E_TOT = 2 * S * N + 2 * EL


def _gemm_body(feat_ref, wctr_ref, wedge_ref, yctr_ref, yedge_ref):
    f = feat_ref[...]
    yctr_ref[...] = lax.dot_general(f, wctr_ref[...], (((1,), (1,)), ((), ())),
                                    preferred_element_type=jnp.float32)
    yedge_ref[...] = jnp.dot(f, wedge_ref[...], preferred_element_type=jnp.float32)


_gemm = pl.pallas_call(
    _gemm_body,
    grid=(GRID,),
    in_specs=[
        pl.BlockSpec((BN, D), lambda i: (i, 0)),
        pl.BlockSpec((D, D), lambda i: (0, 0)),
        pl.BlockSpec((D, NE * D), lambda i: (0, 0)),
    ],
    out_specs=[
        pl.BlockSpec((BN, D), lambda i: (i, 0)),
        pl.BlockSpec((BN, NE * D), lambda i: (i, 0)),
    ],
    out_shape=[
        jax.ShapeDtypeStruct((NP, D), jnp.float32),
        jax.ShapeDtypeStruct((NP, NE * D), jnp.float32),
    ],
)


def _sc_body(yctr, yflat, uloc, gvl, out, uidx, gvidx, rows, sem, acc):
    cid = lax.axis_index("c")
    sid = lax.axis_index("s")
    for cc in range(NCHUNK // 2):
        c = cid * (NCHUNK // 2) + cc
        r0 = c * CH + sid * CHT
        pltpu.sync_copy(yctr.at[pl.ds(r0, CHT)], acc.at[pl.ds(sid * CHT, CHT)])
        pltpu.sync_copy(uloc.at[c * NT + sid], uidx)
        pltpu.sync_copy(gvl.at[c * NT + sid], gvidx)
        plsc.subcore_barrier()

        def body(j, carry):
            pltpu.async_copy(yflat.at[gvidx.at[j]], rows, sem).wait()
            pltpu.sync_copy(rows, acc.at[uidx.at[j]], add=True)
            return carry

        lax.fori_loop(0, NB, body, 0)
        plsc.subcore_barrier()
        pltpu.sync_copy(acc.at[pl.ds(sid * CHT, CHT)], out.at[pl.ds(r0, CHT)])
        plsc.subcore_barrier()


_sc_scatter = pl.kernel(
    _sc_body,
    out_type=jax.ShapeDtypeStruct((NP, D), jnp.float32),
    mesh=plsc.VectorSubcoreMesh(core_axis_name="c", subcore_axis_name="s",
                                num_cores=2, num_subcores=NT),
    scratch_types=[
        pltpu.VMEM((NB, 128), jnp.int32),
        pltpu.VMEM((NB, 128), jnp.int32),
        pltpu.VMEM((128, D), jnp.float32),
        pltpu.SemaphoreType.DMA,
        pltpu.VMEM_SHARED((CH + 16, D), jnp.float32),
    ],
)


def _epi_body(scat_ref, res_ref, w2_ref, g1w_ref, g1b_ref, g2w_ref, g2b_ref, out_ref):
    x = scat_ref[...]
    mu = jnp.mean(x, axis=1, keepdims=True)
    var = jnp.mean((x - mu) * (x - mu), axis=1, keepdims=True)
    a = (x - mu) * lax.rsqrt(var + 1e-5) * g1w_ref[...] + g1b_ref[...]
    a = jnp.maximum(a, 0.0)
    y = lax.dot_general(a, w2_ref[...], (((1,), (1,)), ((), ())),
                        preferred_element_type=jnp.float32)
    mu2 = jnp.mean(y, axis=1, keepdims=True)
    var2 = jnp.mean((y - mu2) * (y - mu2), axis=1, keepdims=True)
    z = (y - mu2) * lax.rsqrt(var2 + 1e-5) * g2w_ref[...] + g2b_ref[...]
    out_ref[...] = jnp.maximum(z + res_ref[...], 0.0)


def _make_epilogue(rows, bn):
    return pl.pallas_call(
        _epi_body,
        grid=(rows // bn,),
        in_specs=[
            pl.BlockSpec((bn, D), lambda i: (i, 0)),
            pl.BlockSpec((bn, D), lambda i: (i, 0)),
            pl.BlockSpec((D, D), lambda i: (0, 0)),
            pl.BlockSpec((1, D), lambda i: (0, 0)),
            pl.BlockSpec((1, D), lambda i: (0, 0)),
            pl.BlockSpec((1, D), lambda i: (0, 0)),
            pl.BlockSpec((1, D), lambda i: (0, 0)),
        ],
        out_specs=pl.BlockSpec((bn, D), lambda i: (i, 0)),
        out_shape=jax.ShapeDtypeStruct((rows, D), jnp.float32),
    )


_epi_mid = _make_epilogue(NP, BN)
_epi_last = _make_epilogue(N, 400)


def _build_edge_schedule(pre_u, pre_v, suc_u, suc_v, left_u, left_v, right_u, right_v):
    u_all = jnp.concatenate([pre_u.reshape(-1), suc_u.reshape(-1), left_u, right_u])
    v_all = jnp.concatenate([pre_v.reshape(-1), suc_v.reshape(-1), left_v, right_v])
    t_all = jnp.concatenate([
        jnp.repeat(jnp.arange(S, dtype=jnp.int32), N),
        jnp.repeat(jnp.arange(S, 2 * S, dtype=jnp.int32), N),
        jnp.full((EL,), 12, jnp.int32),
        jnp.full((EL,), 13, jnp.int32),
    ])
    gv_all = v_all * NE + t_all
    chunk = u_all // CH
    chunk_s, u_s, gv_s = lax.sort((chunk, u_all, gv_all), num_keys=1)
    off = jnp.searchsorted(chunk_s, jnp.arange(NCHUNK, dtype=chunk_s.dtype))
    cnt = jnp.concatenate([off[1:], jnp.array([E_TOT], off.dtype)]) - off
    ct = (cnt + NT - 1) // NT
    c_ix = jnp.arange(NCHUNK, dtype=jnp.int32)[:, None, None]
    j_ix = jnp.arange(NT, dtype=jnp.int32)[None, :, None]
    k_ix = jnp.arange(T_TILE, dtype=jnp.int32)[None, None, :]
    local = j_ix * ct[:, None, None].astype(jnp.int32) + k_ix
    valid = (k_ix < ct[:, None, None]) & (local < cnt[:, None, None])
    g = jnp.clip(off[:, None, None].astype(jnp.int32) + local, 0, E_TOT - 1)
    u_loc = jnp.where(valid, u_s[g] - c_ix * CH, CH).astype(jnp.int32)
    gv_pad = jnp.where(valid, gv_s[g], 0).astype(jnp.int32)
    return (u_loc.reshape(NCHUNK * NT, NB, 128), gv_pad.reshape(NCHUNK * NT, NB, 128))


def kernel(feat, pre_u, pre_v, suc_u, suc_v, left_u, left_v, right_u, right_v,
           W_ctr, W_pre, W_suc, W_left, W_right, W_ctr2,
           gn1_w, gn1_b, gn2_w, gn2_b):
    u_loc, gv_pad = _build_edge_schedule(pre_u, pre_v, suc_u, suc_v,
                                         left_u, left_v, right_u, right_v)
    ws = jnp.concatenate([W_pre, W_suc, W_left[:, None], W_right[:, None]], axis=1)
    wedge = ws.transpose(0, 3, 1, 2).reshape(L, D, NE * D)

    featp = jnp.pad(feat, ((0, NP - N), (0, 0)))
    res = featp
    for i in range(L):
        yctr, yedge = _gemm(featp, W_ctr[i], wedge[i])
        scat = _sc_scatter(yctr, yedge.reshape(NP * NE, D), u_loc, gv_pad)
        epi = _epi_mid if i < L - 1 else _epi_last
        featp = epi(scat, res, W_ctr2[i],
                    gn1_w[i][None], gn1_b[i][None], gn2_w[i][None], gn2_b[i][None])
        res = featp
    return featp
